# Initial kernel scaffold; baseline (speedup 1.0000x reference)
import jax
import jax.numpy as jnp
from jax.experimental import pallas as pl


def kernel(x, W_qkv, b_qkv, W_proj, b_proj, gather_idx):
    raise NotImplementedError("write your pallas kernel here")


# speedup vs baseline: 8.7216x; 8.7216x over previous
import functools

import jax
import jax.numpy as jnp
import numpy as np
from jax.experimental import pallas as pl
from jax.experimental.pallas import tpu as pltpu

_S = 64
_P = 128
_H = 12
_D = 64
_C = _H * _D
_N = _S * _P
_KN = 8


def _qkv_body(x_ref, w_ref, b_ref, q_ref, k_ref, v_ref):
    y = jnp.dot(x_ref[...], w_ref[...], preferred_element_type=jnp.float32)
    y = y + b_ref[...]
    q_ref[...] = y[:, :_C]
    k_ref[...] = y[:, _C:2 * _C]
    v_ref[...] = y[:, 2 * _C:]


def _attn_body(F_ref, m_ref, q_ref, *rest):
    k_refs = rest[0:_KN]
    v_refs = rest[_KN:2 * _KN]
    wp_ref = rest[2 * _KN]
    bp_ref = rest[2 * _KN + 1]
    o_ref = rest[2 * _KN + 2]
    i = pl.program_id(0)
    scale = np.float32(1.0 / np.sqrt(_D))

    ms = [m_ref[i * _KN + c].astype(jnp.float32) for c in range(_KN)]
    q = q_ref[...] * scale
    k_full = jnp.concatenate([k_refs[c][0] for c in range(_KN)], axis=0)
    v_full = jnp.concatenate([ms[c] * v_refs[c][0] for c in range(_KN)],
                             axis=0)

    outs = []
    for h in range(_H):
        sl = slice(h * _D, (h + 1) * _D)
        qh = q[:, sl]
        kh = k_full[:, sl]
        vh = v_full[:, sl]
        s = jax.lax.dot_general(qh, kh, (((1,), (1,)), ((), ())),
                                preferred_element_type=jnp.float32)
        mx = jnp.max(s, axis=1, keepdims=True)
        p = jnp.exp(s - mx)
        denom = ms[0] * jnp.sum(p[:, 0:_P], axis=1)
        for c in range(1, _KN):
            denom = denom + ms[c] * jnp.sum(p[:, c * _P:(c + 1) * _P], axis=1)
        num = jnp.dot(p, vh, preferred_element_type=jnp.float32)
        outs.append(num / denom[:, None])
    o = jnp.concatenate(outs, axis=1)
    o_ref[...] = jnp.dot(o, wp_ref[...],
                         preferred_element_type=jnp.float32) + bp_ref[...]


def _qkv_call(x2, W_qkv, b2):
    blk = 512
    grid = (_N // blk,)
    shp = jax.ShapeDtypeStruct((_N, _C), jnp.float32)
    return pl.pallas_call(
        _qkv_body,
        grid=grid,
        in_specs=[
            pl.BlockSpec((blk, _C), lambda i: (i, 0)),
            pl.BlockSpec((_C, 3 * _C), lambda i: (0, 0)),
            pl.BlockSpec((1, 3 * _C), lambda i: (0, 0)),
        ],
        out_specs=[
            pl.BlockSpec((blk, _C), lambda i: (i, 0)),
            pl.BlockSpec((blk, _C), lambda i: (i, 0)),
            pl.BlockSpec((blk, _C), lambda i: (i, 0)),
        ],
        out_shape=[shp, shp, shp],
    )(x2, W_qkv, b2)


def _attn_call(F_flat, m_flat, q2, k3, v3, W_proj, bp2):
    kv_spec = [
        pl.BlockSpec((1, _P, _C),
                     functools.partial(lambda i, F, m, c: (F[i * _KN + c], 0, 0),
                                       c=c))
        for c in range(_KN)
    ]
    grid_spec = pltpu.PrefetchScalarGridSpec(
        num_scalar_prefetch=2,
        grid=(_S,),
        in_specs=[
            pl.BlockSpec((_P, _C), lambda i, F, m: (i, 0)),
            *kv_spec,
            *kv_spec,
            pl.BlockSpec((_C, _C), lambda i, F, m: (0, 0)),
            pl.BlockSpec((1, _C), lambda i, F, m: (0, 0)),
        ],
        out_specs=pl.BlockSpec((_P, _C), lambda i, F, m: (i, 0)),
    )
    return pl.pallas_call(
        _attn_body,
        grid_spec=grid_spec,
        out_shape=jax.ShapeDtypeStruct((_N, _C), jnp.float32),
    )(F_flat, m_flat, q2, *([k3] * _KN), *([v3] * _KN), W_proj, bp2)


def _prep(gather_idx):
    heads = gather_idx.astype(jnp.int32).reshape(_S, _KN, _P)[:, :, 0] // _P
    ws = jnp.minimum(jnp.min(heads, axis=1), _S - _KN)
    offs = (jnp.arange(_KN, dtype=jnp.int32)[None, :] - ws[:, None]) % _KN
    F = ws[:, None] + offs
    m = jnp.sum(heads[:, None, :] == F[:, :, None], axis=2)
    return F.reshape(-1).astype(jnp.int32), m.reshape(-1).astype(jnp.int32)


def kernel(x, W_qkv, b_qkv, W_proj, b_proj, gather_idx):
    B_, N_, C_ = x.shape
    x2 = x.reshape(N_, C_)
    F_flat, m_flat = _prep(gather_idx)
    q2, k2, v2 = _qkv_call(x2, W_qkv, b_qkv.reshape(1, 3 * _C))
    k3 = k2.reshape(_S, _P, _C)
    v3 = v2.reshape(_S, _P, _C)
    out = _attn_call(F_flat, m_flat, q2, k3, v3, W_proj,
                     b_proj.reshape(1, _C))
    return out.reshape(B_, N_, C_)


# speedup vs baseline: 17.1649x; 1.9681x over previous
import functools

import jax
import jax.numpy as jnp
import numpy as np
from jax.experimental import pallas as pl
from jax.experimental.pallas import tpu as pltpu

_S = 64
_P = 128
_H = 12
_D = 64
_C = _H * _D
_N = _S * _P
_KN = 8
_G = 2
_NS = _KN + _G - 1
_NT = _S // _G
_KEYS = _NS * _P

_QSCALE = np.float32(np.log2(np.e) / np.sqrt(_D))
_LM_SHIFT = np.float32(2.0 ** -20)


def _qkv_body(x_ref, w_ref, b_ref, q_ref, kv_ref):
    y = jnp.dot(x_ref[...], w_ref[...], preferred_element_type=jnp.float32)
    y = y + b_ref[...]
    q_ref[...] = (y[:, :_C] * _QSCALE).astype(jnp.bfloat16)
    kv_ref[...] = y[:, _C:].astype(jnp.bfloat16)


def _attn_body(F_ref, lm_ref, q_ref, *rest):
    kv_refs = rest[0:_NS]
    wp_ref = rest[_NS]
    bp_ref = rest[_NS + 1]
    o_ref = rest[_NS + 2]
    t = pl.program_id(0)

    la = [(lm_ref[(_G * t) * _NS + c].astype(jnp.float32) * _LM_SHIFT
           ).astype(jnp.bfloat16) for c in range(_NS)]
    lb = [(lm_ref[(_G * t + 1) * _NS + c].astype(jnp.float32) * _LM_SHIFT
           ).astype(jnp.bfloat16) for c in range(_NS)]

    zpad = jnp.zeros((_P, _D - _G), jnp.bfloat16)
    kbias = jnp.concatenate(
        [jnp.concatenate(
            [jnp.full((_P, 1), la[c], jnp.bfloat16),
             jnp.full((_P, 1), lb[c], jnp.bfloat16), zpad], axis=1)
         for c in range(_NS)], axis=0)

    row = jax.lax.broadcasted_iota(jnp.int32, (_G * _P, _D), 0)
    col = jax.lax.broadcasted_iota(jnp.int32, (_G * _P, _D), 1)
    e2 = jnp.where((col == 0) & (row < _P), 1.0,
                   jnp.where((col == 1) & (row >= _P), 1.0, 0.0)
                   ).astype(jnp.bfloat16)

    ones_col = jnp.full((_P, _D), 1.0, jnp.bfloat16)
    ones_blk = jnp.concatenate([ones_col] * _NS, axis=0)

    q = q_ref[...]
    q_parts, k_cols, v_cols = [], [], []
    for h in range(_H):
        sl = slice(h * _D, (h + 1) * _D)
        q_parts.append(q[:, sl])
        q_parts.append(e2)
        k_cols.append(jnp.concatenate(
            [kv_refs[c][0][:, sl] for c in range(_NS)], axis=0))
        k_cols.append(kbias)
        v_cols.append(jnp.concatenate(
            [kv_refs[c][0][:, _C + h * _D:_C + (h + 1) * _D]
             for c in range(_NS)], axis=0))
        v_cols.append(ones_blk)
    q_all = jnp.concatenate(q_parts, axis=1)
    k_all = jnp.concatenate(k_cols, axis=1)
    v_all = jnp.concatenate(v_cols, axis=1)

    outs = []
    for h in range(_H):
        sl2 = slice(2 * h * _D, 2 * (h + 1) * _D)
        s = jax.lax.dot_general(q_all[:, sl2], k_all[:, sl2],
                                (((1,), (1,)), ((), ())),
                                preferred_element_type=jnp.float32)
        p = jnp.exp2(s).astype(jnp.bfloat16)
        nd = jnp.dot(p, v_all[:, sl2],
                     preferred_element_type=jnp.float32)
        outs.append(nd[:, :_D] / nd[:, _D:_D + 1])
    o = jnp.concatenate(outs, axis=1).astype(jnp.bfloat16)
    o_ref[...] = jnp.dot(o, wp_ref[...],
                         preferred_element_type=jnp.float32) + bp_ref[...]


def _qkv_call(x2, W_qkv, b2):
    blk = 512
    grid = (_N // blk,)
    shp = jax.ShapeDtypeStruct((_N, _C), jnp.bfloat16)
    return pl.pallas_call(
        _qkv_body,
        grid=grid,
        in_specs=[
            pl.BlockSpec((blk, _C), lambda i: (i, 0)),
            pl.BlockSpec((_C, 3 * _C), lambda i: (0, 0)),
            pl.BlockSpec((1, 3 * _C), lambda i: (0, 0)),
        ],
        out_specs=[
            pl.BlockSpec((blk, _C), lambda i: (i, 0)),
            pl.BlockSpec((blk, 2 * _C), lambda i: (i, 0)),
        ],
        out_shape=[shp, jax.ShapeDtypeStruct((_N, 2 * _C), jnp.bfloat16)],
    )(x2, W_qkv, b2)


def _attn_call(F_flat, lm_flat, q2, kv3, W_proj, bp2):
    kv_spec = [
        pl.BlockSpec((1, _P, 2 * _C),
                     functools.partial(lambda t, F, lm, c: (F[t * _NS + c], 0, 0),
                                       c=c))
        for c in range(_NS)
    ]
    grid_spec = pltpu.PrefetchScalarGridSpec(
        num_scalar_prefetch=2,
        grid=(_NT,),
        in_specs=[
            pl.BlockSpec((_G * _P, _C), lambda t, F, lm: (t, 0)),
            *kv_spec,
            pl.BlockSpec((_C, _C), lambda t, F, lm: (0, 0)),
            pl.BlockSpec((1, _C), lambda t, F, lm: (0, 0)),
        ],
        out_specs=pl.BlockSpec((_G * _P, _C), lambda t, F, lm: (t, 0)),
    )
    return pl.pallas_call(
        _attn_body,
        grid_spec=grid_spec,
        out_shape=jax.ShapeDtypeStruct((_N, _C), jnp.float32),
    )(F_flat, lm_flat, q2, *([kv3] * _NS), W_proj, bp2)


def _prep(gather_idx):
    heads = gather_idx.astype(jnp.int32).reshape(_S, _KN, _P)[:, :, 0] // _P
    ws = jnp.minimum(jnp.min(heads, axis=1), _S - _KN)
    su = jnp.minimum(ws[0::_G], ws[1::_G])
    c9 = jnp.arange(_NS, dtype=jnp.int32)
    nominal = su[:, None] + ((c9[None, :] - su[:, None]) % _NS)
    F = jnp.minimum(nominal, _S - 1)
    nom_f = jnp.repeat(nominal, _G, axis=0)
    m = jnp.sum(heads[:, None, :] == nom_f[:, :, None], axis=2)
    lm = jnp.where(m > 0, jnp.log2(jnp.maximum(m, 1).astype(jnp.float32)),
                   jnp.float32(-1024.0))
    lm_int = jnp.round(lm * np.float32(2.0 ** 20)).astype(jnp.int32)
    return F.reshape(-1).astype(jnp.int32), lm_int.reshape(-1)


def kernel(x, W_qkv, b_qkv, W_proj, b_proj, gather_idx):
    B_, N_, C_ = x.shape
    x2 = x.reshape(N_, C_)
    F_flat, lm_flat = _prep(gather_idx)
    q2, kv2 = _qkv_call(x2, W_qkv, b_qkv.reshape(1, 3 * _C))
    kv3 = kv2.reshape(_S, _P, 2 * _C)
    out = _attn_call(F_flat, lm_flat, q2, kv3,
                     W_proj.astype(jnp.bfloat16), b_proj.reshape(1, _C))
    return out.reshape(B_, N_, C_)
